# baseline (speedup 1.0000x reference)
import math
import functools

import jax
import jax.numpy as jnp
from jax import lax
from jax.experimental import pallas as pl
from jax.experimental.pallas import tpu as pltpu
from jax.experimental.pallas import tpu_sc as plsc

CUTOFF = 10.0

_NC = 2
_NS = 16
_LANES = 16


def _ssp(v):
    return jax.nn.softplus(v) - jnp.log(2.0)


def _make_gather(n_rows_out, d, e_pad):
    nw = _NC * _NS
    per_w = e_pad // nw
    blk = 1024
    n_iter = per_w // blk
    mesh = plsc.VectorSubcoreMesh(core_axis_name="c", subcore_axis_name="s", num_cores=_NC, num_subcores=_NS)

    def body(table_hbm, idx_hbm, out_hbm, idx_v, rows_v, sem):
        wid = lax.axis_index("c") * _NS + lax.axis_index("s")
        base = wid * per_w

        def step(k, _):
            off = pl.multiple_of(base + k * blk, blk)
            pltpu.sync_copy(idx_hbm.at[pl.ds(pl.multiple_of(off // 128, 8), 8)],
                            idx_v)
            for half in range(4):
                for r in range(2):
                    pltpu.async_copy(
                        table_hbm.at[idx_v.at[half * 2 + r]],
                        rows_v.at[pl.ds(r * 128, 128)],
                        sem,
                    ).wait()
                pltpu.sync_copy(rows_v, out_hbm.at[pl.ds(off + half * 256, 256)])
            return 0

        lax.fori_loop(0, n_iter, step, 0)

    return pl.kernel(
        body,
        out_type=jax.ShapeDtypeStruct((e_pad, d), jnp.float32),
        mesh=mesh,
        scratch_types=[
            pltpu.VMEM((8, 128), jnp.int32),
            pltpu.VMEM((256, d), jnp.float32),
            pltpu.SemaphoreType.DMA,
        ],
    )


def _dense_body(ew_ref, attr_ref, xj_ref, w1_ref, b1_ref, w2_ref, b2_ref,
                l1_ref, msg_ref):
    ew = ew_ref[...]
    d2 = jnp.sum(ew * ew, axis=1, keepdims=True)
    d = jnp.sqrt(d2)
    mask = (d <= CUTOFF).astype(jnp.float32)
    cc = 0.5 * (jnp.cos(d * (jnp.pi / CUTOFF)) + 1.0) * mask

    dn = (((1,), (1,)), ((), ()))
    h1 = _ssp(
        lax.dot_general(attr_ref[...], w1_ref[...], dn,
                        preferred_element_type=jnp.float32)
        + b1_ref[...]
    )
    w = (lax.dot_general(h1, w2_ref[...], dn,
                         preferred_element_type=jnp.float32)
         + b2_ref[...]) * cc
    xl = lax.dot_general(xj_ref[...], l1_ref[...], dn,
                         preferred_element_type=jnp.float32)
    msg_ref[...] = w * xl


def _make_dense(e_pad, ng, nf, hid):
    be = 1024
    grid = e_pad // be
    full = lambda shape: pl.BlockSpec(shape, lambda b: (0, 0))
    return pl.pallas_call(
        _dense_body,
        grid=(grid,),
        in_specs=[
            pl.BlockSpec((be, 3), lambda b: (b, 0)),
            pl.BlockSpec((be, ng), lambda b: (b, 0)),
            pl.BlockSpec((be, hid), lambda b: (b, 0)),
            full((nf, ng)),
            full((1, nf)),
            full((nf, nf)),
            full((1, nf)),
            full((nf, hid)),
        ],
        out_specs=pl.BlockSpec((be, nf), lambda b: (b, 0)),
        out_shape=jax.ShapeDtypeStruct((e_pad, nf), jnp.float32),
    )


def _make_scatter(n, nf, e_pad):
    nw = _NC * _NS
    rpt = -(-n // (8 * nw)) * 8
    n_pad = rpt * nw
    dummy = rpt
    blk = 1024
    n_blk = e_pad // blk
    ch = 128
    cap = ch + blk + 16
    i16 = None

    mesh = plsc.VectorSubcoreMesh(core_axis_name="c", subcore_axis_name="s", num_cores=_NC, num_subcores=_NS)

    def body(msg_hbm, idx_hbm, out_hbm, acc_v, rows_v, idx_v, pd_v, pe_v, sem):
        wid = lax.axis_index("c") * _NS + lax.axis_index("s")
        lo = wid * rpt
        iota = lax.iota(jnp.int32, _LANES)

        def zero_row(r, _):
            for l in range(nf // _LANES):
                acc_v[r, pl.ds(l * _LANES, _LANES)] = jnp.zeros(
                    (_LANES,), jnp.float32)
            return 0

        lax.fori_loop(0, rpt + 1, zero_row, 0)
        for g in range(cap // _LANES):
            pe_v[pl.ds(g * _LANES, _LANES)] = jnp.zeros((_LANES,), jnp.int32)

        def accumulate(k, w):
            base = pl.multiple_of(k * ch, ch)
            pltpu.async_copy(msg_hbm.at[pe_v.at[pl.ds(base, ch)]], rows_v,
                             sem).wait()

            def acc_group(g2, _):
                dv = pd_v[pl.ds(base + g2 * _LANES, _LANES)]
                for e in range(_LANES):
                    d = jnp.sum(jnp.where(iota == e, dv, 0))
                    r = g2 * _LANES + e
                    for l in range(nf // _LANES):
                        sl = pl.ds(l * _LANES, _LANES)
                        acc_v[d, sl] = acc_v[d, sl] + rows_v[r, sl]
                return 0

            lax.fori_loop(0, ch // _LANES, acc_group, 0)
            return w

        def scan_block(b, w):
            pltpu.sync_copy(
                idx_hbm.at[pl.ds(pl.multiple_of(b * (blk // 128), 8), 8)],
                idx_v)
            for g in range(blk // _LANES):
                r, lane = g // 8, (g % 8) * _LANES
                v = idx_v[r, pl.ds(lane, _LANES)] - lo
                ok = (v >= 0) & (v < rpt)
                plsc.store_compressed(pd_v.at[pl.ds(w, _LANES)],
                                      jnp.where(ok, v, dummy), mask=ok)
                eidv = jnp.broadcast_to(b * blk + g * _LANES, (_LANES,)) + iota
                plsc.store_compressed(pe_v.at[pl.ds(w, _LANES)], eidv, mask=ok)
                w = w + jnp.sum(ok.astype(jnp.int32))
            nflush = w // ch
            lax.fori_loop(0, nflush, accumulate, w)
            tail = pl.multiple_of(nflush * ch, ch)
            for g in range(ch // _LANES):
                sl_src = pl.ds(tail + g * _LANES, _LANES)
                sl_dst = pl.ds(g * _LANES, _LANES)
                pd_v[sl_dst] = pd_v[sl_src]
                pe_v[sl_dst] = pe_v[sl_src]
            return w - tail

        w = lax.fori_loop(0, n_blk, scan_block, jnp.int32(0))

        for g in range(ch // _LANES):
            pos = g * _LANES + iota
            dv = pd_v[pl.ds(g * _LANES, _LANES)]
            pd_v[pl.ds(g * _LANES, _LANES)] = jnp.where(pos < w, dv, dummy)
        accumulate(0, w)

        pltpu.sync_copy(acc_v.at[pl.ds(0, rpt)],
                        out_hbm.at[pl.ds(pl.multiple_of(wid * rpt, rpt), rpt)])

    return pl.kernel(
        body,
        out_type=jax.ShapeDtypeStruct((n_pad, nf), jnp.float32),
        mesh=mesh,
        scratch_types=[
            pltpu.VMEM((rpt + 1, nf), jnp.float32),
            pltpu.VMEM((ch, nf), jnp.float32),
            pltpu.VMEM((8, 128), jnp.int32),
            pltpu.VMEM((cap,), jnp.int32),
            pltpu.VMEM((cap,), jnp.int32),
            pltpu.SemaphoreType.DMA,
        ],
        compiler_params=pltpu.CompilerParams(needs_layout_passes=False),
    ), n_pad


def _final_body(agg_ref, x_ref, l2_ref, b2_ref, out_ref):
    dn = (((1,), (1,)), ((), ()))
    h = _ssp(
        lax.dot_general(agg_ref[...], l2_ref[...], dn,
                        preferred_element_type=jnp.float32)
        + b2_ref[...]
    )
    out_ref[...] = x_ref[...] + h


def _make_final(n, nf, hid):
    bn = 1000
    grid = n // bn
    return pl.pallas_call(
        _final_body,
        grid=(grid,),
        in_specs=[
            pl.BlockSpec((bn, nf), lambda b: (b, 0)),
            pl.BlockSpec((bn, hid), lambda b: (b, 0)),
            pl.BlockSpec((hid, nf), lambda b: (0, 0)),
            pl.BlockSpec((1, hid), lambda b: (0, 0)),
        ],
        out_specs=pl.BlockSpec((bn, hid), lambda b: (b, 0)),
        out_shape=jax.ShapeDtypeStruct((n, hid), jnp.float32),
    )


@jax.jit
def kernel(x, edge_index, edge_weight, edge_attr, W_mlp1, b_mlp1, W_mlp2,
           b_mlp2, W_lin1, W_lin2, b_lin2):
    n, hid = x.shape
    e = edge_index.shape[1]
    nf, ng = W_mlp1.shape

    unit = _NC * _NS * 256
    e_pad = -(-e // unit) * unit
    pad = e_pad - e

    i_idx = edge_index[0].astype(jnp.int32)
    j_idx = edge_index[1].astype(jnp.int32)
    ip = jnp.pad(i_idx, (0, pad),
                 constant_values=1 << 29).reshape(e_pad // 128, 128)
    jp = jnp.concatenate(
        [j_idx, jnp.arange(pad, dtype=jnp.int32) % n]
    ).reshape(e_pad // 128, 128)
    ew = jnp.pad(edge_weight, ((0, pad), (0, 0)))
    attr = jnp.pad(edge_attr, ((0, pad), (0, 0)))

    xj = _make_gather(n, hid, e_pad)(x, jp)
    msg = _make_dense(e_pad, ng, nf, hid)(
        ew, attr, xj, W_mlp1, b_mlp1.reshape(1, nf), W_mlp2,
        b_mlp2.reshape(1, nf), W_lin1)
    scatter, n_pad = _make_scatter(n, nf, e_pad)
    agg = scatter(msg, ip)[:n]
    return _make_final(n, nf, hid)(agg, x, W_lin2, b_lin2.reshape(1, hid))


# speedup vs baseline: 1.0438x; 1.0438x over previous
import math
import functools

import jax
import jax.numpy as jnp
from jax import lax
from jax.experimental import pallas as pl
from jax.experimental.pallas import tpu as pltpu
from jax.experimental.pallas import tpu_sc as plsc

CUTOFF = 10.0

_NC = 2
_NS = 16
_LANES = 16


def _ssp(v):
    return jax.nn.softplus(v) - jnp.log(2.0)


def _make_gather(n_rows_out, d, e_pad):
    nw = _NC * _NS
    per_w = e_pad // nw
    blk = 1024
    n_iter = per_w // blk
    mesh = plsc.VectorSubcoreMesh(core_axis_name="c", subcore_axis_name="s", num_cores=_NC, num_subcores=_NS)

    def body(table_hbm, idx_hbm, out_hbm, idx_v, rows_v, sem):
        wid = lax.axis_index("c") * _NS + lax.axis_index("s")
        base = wid * per_w

        def step(k, _):
            off = pl.multiple_of(base + k * blk, blk)
            pltpu.sync_copy(idx_hbm.at[pl.ds(pl.multiple_of(off // 128, 8), 8)],
                            idx_v)
            for half in range(4):
                for r in range(2):
                    pltpu.async_copy(
                        table_hbm.at[idx_v.at[half * 2 + r]],
                        rows_v.at[pl.ds(r * 128, 128)],
                        sem,
                    ).wait()
                pltpu.sync_copy(rows_v, out_hbm.at[pl.ds(off + half * 256, 256)])
            return 0

        lax.fori_loop(0, n_iter, step, 0)

    return pl.kernel(
        body,
        out_type=jax.ShapeDtypeStruct((e_pad, d), jnp.float32),
        mesh=mesh,
        scratch_types=[
            pltpu.VMEM((8, 128), jnp.int32),
            pltpu.VMEM((256, d), jnp.float32),
            pltpu.SemaphoreType.DMA,
        ],
    )


def _dense_body(ew_ref, attr_ref, xj_ref, w1_ref, b1_ref, w2_ref, b2_ref,
                l1_ref, msg_ref):
    ew = ew_ref[...]
    d2 = jnp.sum(ew * ew, axis=1, keepdims=True)
    d = jnp.sqrt(d2)
    mask = (d <= CUTOFF).astype(jnp.float32)
    cc = 0.5 * (jnp.cos(d * (jnp.pi / CUTOFF)) + 1.0) * mask

    dn = (((1,), (1,)), ((), ()))
    h1 = _ssp(
        lax.dot_general(attr_ref[...], w1_ref[...], dn,
                        preferred_element_type=jnp.float32)
        + b1_ref[...]
    )
    w = (lax.dot_general(h1, w2_ref[...], dn,
                         preferred_element_type=jnp.float32)
         + b2_ref[...]) * cc
    xl = lax.dot_general(xj_ref[...], l1_ref[...], dn,
                         preferred_element_type=jnp.float32)
    msg_ref[...] = w * xl


def _make_dense(e_pad, ng, nf, hid):
    be = 1024
    grid = e_pad // be
    full = lambda shape: pl.BlockSpec(shape, lambda b: (0, 0))
    return pl.pallas_call(
        _dense_body,
        grid=(grid,),
        in_specs=[
            pl.BlockSpec((be, 3), lambda b: (b, 0)),
            pl.BlockSpec((be, ng), lambda b: (b, 0)),
            pl.BlockSpec((be, hid), lambda b: (b, 0)),
            full((nf, ng)),
            full((1, nf)),
            full((nf, nf)),
            full((1, nf)),
            full((nf, hid)),
        ],
        out_specs=pl.BlockSpec((be, nf), lambda b: (b, 0)),
        out_shape=jax.ShapeDtypeStruct((e_pad, nf), jnp.float32),
    )


def _make_scatter(n, nf, e_pad):
    nw = _NC * _NS
    rpt = -(-n // (8 * nw)) * 8
    n_pad = rpt * nw
    dummy = rpt
    blk = 1024
    n_blk = e_pad // blk
    ch = 128
    cap = ch + blk + 16
    i16 = None

    mesh = plsc.VectorSubcoreMesh(core_axis_name="c", subcore_axis_name="s", num_cores=_NC, num_subcores=_NS)

    def body(msg_hbm, idx_hbm, out_hbm, acc_v, rows_v, idx_v, pd_v, pe_v, sem):
        wid = lax.axis_index("c") * _NS + lax.axis_index("s")
        lo = wid * rpt
        iota = lax.iota(jnp.int32, _LANES)

        def zero_row(r, _):
            for l in range(nf // _LANES):
                acc_v[r, pl.ds(l * _LANES, _LANES)] = jnp.zeros(
                    (_LANES,), jnp.float32)
            return 0

        lax.fori_loop(0, rpt + 1, zero_row, 0)
        for g in range(cap // _LANES):
            pe_v[pl.ds(g * _LANES, _LANES)] = jnp.zeros((_LANES,), jnp.int32)

        def accumulate(k, w):
            base = pl.multiple_of(k * ch, ch)
            pltpu.async_copy(msg_hbm.at[pe_v.at[pl.ds(base, ch)]], rows_v,
                             sem).wait()

            def acc_group(g2, _):
                dv = pd_v[pl.ds(base + g2 * _LANES, _LANES)]
                for e in range(_LANES):
                    d = jnp.sum(jnp.where(iota == e, dv, 0))
                    r = g2 * _LANES + e
                    for l in range(nf // _LANES):
                        sl = pl.ds(l * _LANES, _LANES)
                        acc_v[d, sl] = acc_v[d, sl] + rows_v[r, sl]
                return 0

            lax.fori_loop(0, ch // _LANES, acc_group, 0)
            return w

        def scan_block(b, w):
            pltpu.sync_copy(
                idx_hbm.at[pl.ds(pl.multiple_of(b * (blk // 128), 8), 8)],
                idx_v)
            for gb in range(0, blk // _LANES, 8):
                vals = []
                for g in range(gb, gb + 8):
                    r, lane = g // 8, (g % 8) * _LANES
                    v = idx_v[r, pl.ds(lane, _LANES)] - lo
                    ok = (v >= 0) & (v < rpt)
                    vals.append((v, ok, jnp.sum(ok.astype(jnp.int32))))
                for g in range(gb, gb + 8):
                    v, ok, cnt = vals[g - gb]
                    plsc.store_compressed(pd_v.at[pl.ds(w, _LANES)],
                                          jnp.where(ok, v, dummy), mask=ok)
                    eidv = jnp.broadcast_to(b * blk + g * _LANES,
                                            (_LANES,)) + iota
                    plsc.store_compressed(pe_v.at[pl.ds(w, _LANES)], eidv,
                                          mask=ok)
                    w = w + cnt
            nflush = w // ch
            lax.fori_loop(0, nflush, accumulate, w)
            tail = pl.multiple_of(nflush * ch, ch)
            for g in range(ch // _LANES):
                sl_src = pl.ds(tail + g * _LANES, _LANES)
                sl_dst = pl.ds(g * _LANES, _LANES)
                pd_v[sl_dst] = pd_v[sl_src]
                pe_v[sl_dst] = pe_v[sl_src]
            return w - tail

        w = lax.fori_loop(0, n_blk, scan_block, jnp.int32(0))

        for g in range(ch // _LANES):
            pos = g * _LANES + iota
            dv = pd_v[pl.ds(g * _LANES, _LANES)]
            pd_v[pl.ds(g * _LANES, _LANES)] = jnp.where(pos < w, dv, dummy)
        accumulate(0, w)

        pltpu.sync_copy(acc_v.at[pl.ds(0, rpt)],
                        out_hbm.at[pl.ds(pl.multiple_of(wid * rpt, rpt), rpt)])

    return pl.kernel(
        body,
        out_type=jax.ShapeDtypeStruct((n_pad, nf), jnp.float32),
        mesh=mesh,
        scratch_types=[
            pltpu.VMEM((rpt + 1, nf), jnp.float32),
            pltpu.VMEM((ch, nf), jnp.float32),
            pltpu.VMEM((8, 128), jnp.int32),
            pltpu.VMEM((cap,), jnp.int32),
            pltpu.VMEM((cap,), jnp.int32),
            pltpu.SemaphoreType.DMA,
        ],
        compiler_params=pltpu.CompilerParams(needs_layout_passes=False),
    ), n_pad


def _final_body(agg_ref, x_ref, l2_ref, b2_ref, out_ref):
    dn = (((1,), (1,)), ((), ()))
    h = _ssp(
        lax.dot_general(agg_ref[...], l2_ref[...], dn,
                        preferred_element_type=jnp.float32)
        + b2_ref[...]
    )
    out_ref[...] = x_ref[...] + h


def _make_final(n, nf, hid):
    bn = 1000
    grid = n // bn
    return pl.pallas_call(
        _final_body,
        grid=(grid,),
        in_specs=[
            pl.BlockSpec((bn, nf), lambda b: (b, 0)),
            pl.BlockSpec((bn, hid), lambda b: (b, 0)),
            pl.BlockSpec((hid, nf), lambda b: (0, 0)),
            pl.BlockSpec((1, hid), lambda b: (0, 0)),
        ],
        out_specs=pl.BlockSpec((bn, hid), lambda b: (b, 0)),
        out_shape=jax.ShapeDtypeStruct((n, hid), jnp.float32),
    )


@jax.jit
def kernel(x, edge_index, edge_weight, edge_attr, W_mlp1, b_mlp1, W_mlp2,
           b_mlp2, W_lin1, W_lin2, b_lin2):
    n, hid = x.shape
    e = edge_index.shape[1]
    nf, ng = W_mlp1.shape

    unit = _NC * _NS * 256
    e_pad = -(-e // unit) * unit
    pad = e_pad - e

    i_idx = edge_index[0].astype(jnp.int32)
    j_idx = edge_index[1].astype(jnp.int32)
    ip = jnp.pad(i_idx, (0, pad),
                 constant_values=1 << 29).reshape(e_pad // 128, 128)
    jp = jnp.concatenate(
        [j_idx, jnp.arange(pad, dtype=jnp.int32) % n]
    ).reshape(e_pad // 128, 128)
    ew = jnp.pad(edge_weight, ((0, pad), (0, 0)))
    attr = jnp.pad(edge_attr, ((0, pad), (0, 0)))

    xj = _make_gather(n, hid, e_pad)(x, jp)
    msg = _make_dense(e_pad, ng, nf, hid)(
        ew, attr, xj, W_mlp1, b_mlp1.reshape(1, nf), W_mlp2,
        b_mlp2.reshape(1, nf), W_lin1)
    scatter, n_pad = _make_scatter(n, nf, e_pad)
    agg = scatter(msg, ip)[:n]
    return _make_final(n, nf, hid)(agg, x, W_lin2, b_lin2.reshape(1, hid))


# speedup vs baseline: 1.0440x; 1.0002x over previous
import math
import functools

import jax
import jax.numpy as jnp
from jax import lax
from jax.experimental import pallas as pl
from jax.experimental.pallas import tpu as pltpu
from jax.experimental.pallas import tpu_sc as plsc

CUTOFF = 10.0

_NC = 2
_NS = 16
_LANES = 16


def _ssp(v):
    return jax.nn.softplus(v) - jnp.log(2.0)


def _make_gather(n_rows_out, d, e_pad):
    nw = _NC * _NS
    per_w = e_pad // nw
    blk = 1024
    n_iter = per_w // blk
    mesh = plsc.VectorSubcoreMesh(core_axis_name="c", subcore_axis_name="s", num_cores=_NC, num_subcores=_NS)

    def body(table_hbm, idx_hbm, out_hbm, idx_v, rows_v, sem):
        wid = lax.axis_index("c") * _NS + lax.axis_index("s")
        base = wid * per_w

        def step(k, _):
            off = pl.multiple_of(base + k * blk, blk)
            pltpu.sync_copy(idx_hbm.at[pl.ds(pl.multiple_of(off // 128, 8), 8)],
                            idx_v)
            for half in range(4):
                for r in range(2):
                    pltpu.async_copy(
                        table_hbm.at[idx_v.at[half * 2 + r]],
                        rows_v.at[pl.ds(r * 128, 128)],
                        sem,
                    ).wait()
                pltpu.sync_copy(rows_v, out_hbm.at[pl.ds(off + half * 256, 256)])
            return 0

        lax.fori_loop(0, n_iter, step, 0)

    return pl.kernel(
        body,
        out_type=jax.ShapeDtypeStruct((e_pad, d), jnp.float32),
        mesh=mesh,
        scratch_types=[
            pltpu.VMEM((8, 128), jnp.int32),
            pltpu.VMEM((256, d), jnp.float32),
            pltpu.SemaphoreType.DMA,
        ],
    )


def _dense_body(ew_ref, attr_ref, xj_ref, w1_ref, b1_ref, w2_ref, b2_ref,
                l1_ref, msg_ref):
    ew = ew_ref[...]
    d2 = jnp.sum(ew * ew, axis=1, keepdims=True)
    d = jnp.sqrt(d2)
    mask = (d <= CUTOFF).astype(jnp.float32)
    cc = 0.5 * (jnp.cos(d * (jnp.pi / CUTOFF)) + 1.0) * mask

    dn = (((1,), (1,)), ((), ()))
    bf = jnp.bfloat16
    h1 = _ssp(
        lax.dot_general(attr_ref[...].astype(bf), w1_ref[...].astype(bf), dn,
                        preferred_element_type=jnp.float32)
        + b1_ref[...]
    )
    w = (lax.dot_general(h1.astype(bf), w2_ref[...].astype(bf), dn,
                         preferred_element_type=jnp.float32)
         + b2_ref[...]) * cc
    xl = lax.dot_general(xj_ref[...].astype(bf), l1_ref[...].astype(bf), dn,
                         preferred_element_type=jnp.float32)
    msg_ref[...] = w * xl


def _make_dense(e_pad, ng, nf, hid):
    be = 1024
    grid = e_pad // be
    full = lambda shape: pl.BlockSpec(shape, lambda b: (0, 0))
    return pl.pallas_call(
        _dense_body,
        grid=(grid,),
        in_specs=[
            pl.BlockSpec((be, 3), lambda b: (b, 0)),
            pl.BlockSpec((be, ng), lambda b: (b, 0)),
            pl.BlockSpec((be, hid), lambda b: (b, 0)),
            full((nf, ng)),
            full((1, nf)),
            full((nf, nf)),
            full((1, nf)),
            full((nf, hid)),
        ],
        out_specs=pl.BlockSpec((be, nf), lambda b: (b, 0)),
        out_shape=jax.ShapeDtypeStruct((e_pad, nf), jnp.float32),
    )


def _make_scatter(n, nf, e_pad):
    nw = _NC * _NS
    rpt = -(-n // (8 * nw)) * 8
    n_pad = rpt * nw
    dummy = rpt
    blk = 1024
    n_blk = e_pad // blk
    ch = 128
    cap = ch + blk + 16
    i16 = None

    mesh = plsc.VectorSubcoreMesh(core_axis_name="c", subcore_axis_name="s", num_cores=_NC, num_subcores=_NS)

    def body(msg_hbm, idx_hbm, out_hbm, acc_v, rows_v, idx_v, pd_v, pe_v, sem):
        wid = lax.axis_index("c") * _NS + lax.axis_index("s")
        lo = wid * rpt
        iota = lax.iota(jnp.int32, _LANES)

        def zero_row(r, _):
            for l in range(nf // _LANES):
                acc_v[r, pl.ds(l * _LANES, _LANES)] = jnp.zeros(
                    (_LANES,), jnp.float32)
            return 0

        lax.fori_loop(0, rpt + 1, zero_row, 0)
        for g in range(cap // _LANES):
            pe_v[pl.ds(g * _LANES, _LANES)] = jnp.zeros((_LANES,), jnp.int32)

        def accumulate(k, w):
            base = pl.multiple_of(k * ch, ch)
            pltpu.async_copy(msg_hbm.at[pe_v.at[pl.ds(base, ch)]], rows_v,
                             sem).wait()

            def acc_group(g2, _):
                dv = pd_v[pl.ds(base + g2 * _LANES, _LANES)]
                for e in range(_LANES):
                    d = jnp.sum(jnp.where(iota == e, dv, 0))
                    r = g2 * _LANES + e
                    for l in range(nf // _LANES):
                        sl = pl.ds(l * _LANES, _LANES)
                        acc_v[d, sl] = acc_v[d, sl] + rows_v[r, sl]
                return 0

            lax.fori_loop(0, ch // _LANES, acc_group, 0)
            return w

        def scan_block(b, w):
            pltpu.sync_copy(
                idx_hbm.at[pl.ds(pl.multiple_of(b * (blk // 128), 8), 8)],
                idx_v)
            for gb in range(0, blk // _LANES, 8):
                vals = []
                for g in range(gb, gb + 8):
                    r, lane = g // 8, (g % 8) * _LANES
                    v = idx_v[r, pl.ds(lane, _LANES)] - lo
                    ok = (v >= 0) & (v < rpt)
                    vals.append((v, ok, jnp.sum(ok.astype(jnp.int32))))
                for g in range(gb, gb + 8):
                    v, ok, cnt = vals[g - gb]
                    plsc.store_compressed(pd_v.at[pl.ds(w, _LANES)],
                                          jnp.where(ok, v, dummy), mask=ok)
                    eidv = jnp.broadcast_to(b * blk + g * _LANES,
                                            (_LANES,)) + iota
                    plsc.store_compressed(pe_v.at[pl.ds(w, _LANES)], eidv,
                                          mask=ok)
                    w = w + cnt
            nflush = w // ch
            lax.fori_loop(0, nflush, accumulate, w)
            tail = pl.multiple_of(nflush * ch, ch)
            for g in range(ch // _LANES):
                sl_src = pl.ds(tail + g * _LANES, _LANES)
                sl_dst = pl.ds(g * _LANES, _LANES)
                pd_v[sl_dst] = pd_v[sl_src]
                pe_v[sl_dst] = pe_v[sl_src]
            return w - tail

        w = lax.fori_loop(0, n_blk, scan_block, jnp.int32(0))

        for g in range(ch // _LANES):
            pos = g * _LANES + iota
            dv = pd_v[pl.ds(g * _LANES, _LANES)]
            pd_v[pl.ds(g * _LANES, _LANES)] = jnp.where(pos < w, dv, dummy)
        accumulate(0, w)

        pltpu.sync_copy(acc_v.at[pl.ds(0, rpt)],
                        out_hbm.at[pl.ds(pl.multiple_of(wid * rpt, rpt), rpt)])

    return pl.kernel(
        body,
        out_type=jax.ShapeDtypeStruct((n_pad, nf), jnp.float32),
        mesh=mesh,
        scratch_types=[
            pltpu.VMEM((rpt + 1, nf), jnp.float32),
            pltpu.VMEM((ch, nf), jnp.float32),
            pltpu.VMEM((8, 128), jnp.int32),
            pltpu.VMEM((cap,), jnp.int32),
            pltpu.VMEM((cap,), jnp.int32),
            pltpu.SemaphoreType.DMA,
        ],
        compiler_params=pltpu.CompilerParams(needs_layout_passes=False),
    ), n_pad


def _final_body(agg_ref, x_ref, l2_ref, b2_ref, out_ref):
    dn = (((1,), (1,)), ((), ()))
    h = _ssp(
        lax.dot_general(agg_ref[...], l2_ref[...], dn,
                        preferred_element_type=jnp.float32)
        + b2_ref[...]
    )
    out_ref[...] = x_ref[...] + h


def _make_final(n, nf, hid):
    bn = 1000
    grid = n // bn
    return pl.pallas_call(
        _final_body,
        grid=(grid,),
        in_specs=[
            pl.BlockSpec((bn, nf), lambda b: (b, 0)),
            pl.BlockSpec((bn, hid), lambda b: (b, 0)),
            pl.BlockSpec((hid, nf), lambda b: (0, 0)),
            pl.BlockSpec((1, hid), lambda b: (0, 0)),
        ],
        out_specs=pl.BlockSpec((bn, hid), lambda b: (b, 0)),
        out_shape=jax.ShapeDtypeStruct((n, hid), jnp.float32),
    )


@jax.jit
def kernel(x, edge_index, edge_weight, edge_attr, W_mlp1, b_mlp1, W_mlp2,
           b_mlp2, W_lin1, W_lin2, b_lin2):
    n, hid = x.shape
    e = edge_index.shape[1]
    nf, ng = W_mlp1.shape

    unit = _NC * _NS * 256
    e_pad = -(-e // unit) * unit
    pad = e_pad - e

    i_idx = edge_index[0].astype(jnp.int32)
    j_idx = edge_index[1].astype(jnp.int32)
    ip = jnp.pad(i_idx, (0, pad),
                 constant_values=1 << 29).reshape(e_pad // 128, 128)
    jp = jnp.concatenate(
        [j_idx, jnp.arange(pad, dtype=jnp.int32) % n]
    ).reshape(e_pad // 128, 128)
    ew = jnp.pad(edge_weight, ((0, pad), (0, 0)))
    attr = jnp.pad(edge_attr, ((0, pad), (0, 0)))

    xj = _make_gather(n, hid, e_pad)(x, jp)
    msg = _make_dense(e_pad, ng, nf, hid)(
        ew, attr, xj, W_mlp1, b_mlp1.reshape(1, nf), W_mlp2,
        b_mlp2.reshape(1, nf), W_lin1)
    scatter, n_pad = _make_scatter(n, nf, e_pad)
    agg = scatter(msg, ip)[:n]
    return _make_final(n, nf, hid)(agg, x, W_lin2, b_lin2.reshape(1, hid))


# speedup vs baseline: 1.1152x; 1.0682x over previous
import math
import functools

import jax
import jax.numpy as jnp
from jax import lax
from jax.experimental import pallas as pl
from jax.experimental.pallas import tpu as pltpu
from jax.experimental.pallas import tpu_sc as plsc

CUTOFF = 10.0

_NC = 2
_NS = 16
_LANES = 16


def _ssp(v):
    return jax.nn.softplus(v) - jnp.log(2.0)


def _make_gather(n_rows_out, d, e_pad):
    nw = _NC * _NS
    per_w = e_pad // nw
    blk = 1024
    n_iter = per_w // blk
    mesh = plsc.VectorSubcoreMesh(core_axis_name="c", subcore_axis_name="s", num_cores=_NC, num_subcores=_NS)

    def body(table_hbm, idx_hbm, out_hbm, idx_v, rows_v, gsem, wsem):
        wid = lax.axis_index("c") * _NS + lax.axis_index("s")
        base = wid * per_w

        def step(b, _):
            off = pl.multiple_of(base + b * blk, blk)
            pltpu.sync_copy(idx_hbm.at[pl.ds(pl.multiple_of(off // 128, 8), 8)],
                            idx_v)
            for i in range(8):
                slot = i & 1
                if i >= 2:
                    pltpu.make_async_copy(
                        rows_v.at[slot],
                        out_hbm.at[pl.ds(off, 128)], wsem).wait()
                else:
                    @pl.when(b > 0)
                    def _():
                        pltpu.make_async_copy(
                            rows_v.at[slot],
                            out_hbm.at[pl.ds(off, 128)], wsem).wait()
                pltpu.async_copy(table_hbm.at[idx_v.at[i]],
                                 rows_v.at[slot], gsem).wait()
                pltpu.async_copy(rows_v.at[slot],
                                 out_hbm.at[pl.ds(off + i * 128, 128)], wsem)
            return 0

        lax.fori_loop(0, n_iter, step, 0)
        for _ in range(2):
            pltpu.make_async_copy(rows_v.at[0],
                                  out_hbm.at[pl.ds(0, 128)], wsem).wait()

    return pl.kernel(
        body,
        out_type=jax.ShapeDtypeStruct((e_pad, d), jnp.float32),
        mesh=mesh,
        scratch_types=[
            pltpu.VMEM((8, 128), jnp.int32),
            pltpu.VMEM((2, 128, d), jnp.float32),
            pltpu.SemaphoreType.DMA,
            pltpu.SemaphoreType.DMA,
        ],
    )


def _dense_body(ew_ref, attr_ref, xj_ref, w1_ref, b1_ref, w2_ref, b2_ref,
                l1_ref, msg_ref):
    ew = ew_ref[...]
    d2 = jnp.sum(ew * ew, axis=1, keepdims=True)
    d = jnp.sqrt(d2)
    mask = (d <= CUTOFF).astype(jnp.float32)
    cc = 0.5 * (jnp.cos(d * (jnp.pi / CUTOFF)) + 1.0) * mask

    dn = (((1,), (1,)), ((), ()))
    bf = jnp.bfloat16
    h1 = _ssp(
        lax.dot_general(attr_ref[...].astype(bf), w1_ref[...].astype(bf), dn,
                        preferred_element_type=jnp.float32)
        + b1_ref[...]
    )
    w = (lax.dot_general(h1.astype(bf), w2_ref[...].astype(bf), dn,
                         preferred_element_type=jnp.float32)
         + b2_ref[...]) * cc
    xl = lax.dot_general(xj_ref[...].astype(bf), l1_ref[...].astype(bf), dn,
                         preferred_element_type=jnp.float32)
    msg_ref[...] = w * xl


def _make_dense(e_pad, ng, nf, hid):
    be = 1024
    grid = e_pad // be
    full = lambda shape: pl.BlockSpec(shape, lambda b: (0, 0))
    return pl.pallas_call(
        _dense_body,
        grid=(grid,),
        in_specs=[
            pl.BlockSpec((be, 3), lambda b: (b, 0)),
            pl.BlockSpec((be, ng), lambda b: (b, 0)),
            pl.BlockSpec((be, hid), lambda b: (b, 0)),
            full((nf, ng)),
            full((1, nf)),
            full((nf, nf)),
            full((1, nf)),
            full((nf, hid)),
        ],
        out_specs=pl.BlockSpec((be, nf), lambda b: (b, 0)),
        out_shape=jax.ShapeDtypeStruct((e_pad, nf), jnp.float32),
    )


def _make_scatter(n, nf, e_pad):
    nw = _NC * _NS
    rpt = -(-n // (8 * nw)) * 8
    n_pad = rpt * nw
    dummy = rpt
    blk = 1024
    n_blk = e_pad // blk
    ch = 64
    cap = ch + blk + 16

    mesh = plsc.VectorSubcoreMesh(core_axis_name="c", subcore_axis_name="s", num_cores=_NC, num_subcores=_NS)

    def body(msg_hbm, idx_hbm, out_hbm, acc_v, rows_v, idx_v, pd_v, pe_v,
             gsem, isem):
        wid = lax.axis_index("c") * _NS + lax.axis_index("s")
        lo = wid * rpt
        iota = lax.iota(jnp.int32, _LANES)

        def zero_row(r, _):
            for l in range(nf // _LANES):
                acc_v[r, pl.ds(l * _LANES, _LANES)] = jnp.zeros(
                    (_LANES,), jnp.float32)
            return 0

        lax.fori_loop(0, rpt + 1, zero_row, 0)
        for g in range(cap // _LANES):
            pe_v[pl.ds(g * _LANES, _LANES)] = jnp.zeros((_LANES,), jnp.int32)

        def acc_chunk(slot, base):
            def acc_group(g2, _):
                dv = pd_v[pl.ds(base + g2 * _LANES, _LANES)]
                for e in range(_LANES):
                    d = jnp.sum(jnp.where(iota == e, dv, 0))
                    r = g2 * _LANES + e
                    for l in range(nf // _LANES):
                        sl = pl.ds(l * _LANES, _LANES)
                        acc_v[d, sl] = acc_v[d, sl] + rows_v[slot, r, sl]
                return 0

            lax.fori_loop(0, ch // _LANES, acc_group, 0)

        def scan_block(b, w):
            bslot = b & 1
            pltpu.make_async_copy(
                idx_hbm.at[pl.ds(0, 8)], idx_v.at[bslot], isem).wait()

            @pl.when(b + 1 < n_blk)
            def _():
                pltpu.async_copy(
                    idx_hbm.at[pl.ds(
                        pl.multiple_of((b + 1) * (blk // 128), 8), 8)],
                    idx_v.at[(b + 1) & 1], isem)
            for gb in range(0, blk // _LANES, 8):
                vals = []
                for g in range(gb, gb + 8):
                    r, lane = g // 8, (g % 8) * _LANES
                    v = idx_v[bslot, r, pl.ds(lane, _LANES)] - lo
                    ok = (v >= 0) & (v < rpt)
                    vals.append((v, ok, jnp.sum(ok.astype(jnp.int32))))
                for g in range(gb, gb + 8):
                    v, ok, cnt = vals[g - gb]
                    plsc.store_compressed(pd_v.at[pl.ds(w, _LANES)],
                                          jnp.where(ok, v, dummy), mask=ok)
                    eidv = jnp.broadcast_to(b * blk + g * _LANES,
                                            (_LANES,)) + iota
                    plsc.store_compressed(pe_v.at[pl.ds(w, _LANES)], eidv,
                                          mask=ok)
                    w = w + cnt
            nflush = w // ch

            @pl.when(nflush > 0)
            def _():
                pltpu.async_copy(msg_hbm.at[pe_v.at[pl.ds(0, ch)]],
                                 rows_v.at[0], gsem)

            def flush(k, w):
                pltpu.make_async_copy(msg_hbm.at[pe_v.at[pl.ds(0, ch)]],
                                      rows_v.at[0], gsem).wait()

                @pl.when(k + 1 < nflush)
                def _():
                    pltpu.async_copy(
                        msg_hbm.at[pe_v.at[pl.ds(
                            pl.multiple_of((k + 1) * ch, ch), ch)]],
                        rows_v.at[(k + 1) & 1], gsem)

                acc_chunk(k & 1, pl.multiple_of(k * ch, ch))
                return w

            lax.fori_loop(0, nflush, flush, w)
            tail = pl.multiple_of(nflush * ch, ch)
            for g in range(ch // _LANES):
                sl_src = pl.ds(tail + g * _LANES, _LANES)
                sl_dst = pl.ds(g * _LANES, _LANES)
                pd_v[sl_dst] = pd_v[sl_src]
                pe_v[sl_dst] = pe_v[sl_src]
            return w - tail

        pltpu.async_copy(idx_hbm.at[pl.ds(0, 8)], idx_v.at[0], isem)
        w = lax.fori_loop(0, n_blk, scan_block, jnp.int32(0))

        for g in range(ch // _LANES):
            pos = g * _LANES + iota
            dv = pd_v[pl.ds(g * _LANES, _LANES)]
            pd_v[pl.ds(g * _LANES, _LANES)] = jnp.where(pos < w, dv, dummy)
        pltpu.async_copy(msg_hbm.at[pe_v.at[pl.ds(0, ch)]], rows_v.at[0],
                         gsem).wait()
        acc_chunk(0, 0)

        pltpu.sync_copy(acc_v.at[pl.ds(0, rpt)],
                        out_hbm.at[pl.ds(pl.multiple_of(wid * rpt, rpt), rpt)])

    return pl.kernel(
        body,
        out_type=jax.ShapeDtypeStruct((n_pad, nf), jnp.float32),
        mesh=mesh,
        scratch_types=[
            pltpu.VMEM((rpt + 1, nf), jnp.float32),
            pltpu.VMEM((2, ch, nf), jnp.float32),
            pltpu.VMEM((2, 8, 128), jnp.int32),
            pltpu.VMEM((cap,), jnp.int32),
            pltpu.VMEM((cap,), jnp.int32),
            pltpu.SemaphoreType.DMA,
            pltpu.SemaphoreType.DMA,
        ],
        compiler_params=pltpu.CompilerParams(needs_layout_passes=False),
    ), n_pad


def _final_body(agg_ref, x_ref, l2_ref, b2_ref, out_ref):
    dn = (((1,), (1,)), ((), ()))
    h = _ssp(
        lax.dot_general(agg_ref[...], l2_ref[...], dn,
                        preferred_element_type=jnp.float32)
        + b2_ref[...]
    )
    out_ref[...] = x_ref[...] + h


def _make_final(n, nf, hid):
    bn = 1000
    grid = n // bn
    return pl.pallas_call(
        _final_body,
        grid=(grid,),
        in_specs=[
            pl.BlockSpec((bn, nf), lambda b: (b, 0)),
            pl.BlockSpec((bn, hid), lambda b: (b, 0)),
            pl.BlockSpec((hid, nf), lambda b: (0, 0)),
            pl.BlockSpec((1, hid), lambda b: (0, 0)),
        ],
        out_specs=pl.BlockSpec((bn, hid), lambda b: (b, 0)),
        out_shape=jax.ShapeDtypeStruct((n, hid), jnp.float32),
    )


@jax.jit
def kernel(x, edge_index, edge_weight, edge_attr, W_mlp1, b_mlp1, W_mlp2,
           b_mlp2, W_lin1, W_lin2, b_lin2):
    n, hid = x.shape
    e = edge_index.shape[1]
    nf, ng = W_mlp1.shape

    unit = _NC * _NS * 256
    e_pad = -(-e // unit) * unit
    pad = e_pad - e

    i_idx = edge_index[0].astype(jnp.int32)
    j_idx = edge_index[1].astype(jnp.int32)
    ip = jnp.pad(i_idx, (0, pad),
                 constant_values=1 << 29).reshape(e_pad // 128, 128)
    jp = jnp.concatenate(
        [j_idx, jnp.arange(pad, dtype=jnp.int32) % n]
    ).reshape(e_pad // 128, 128)
    ew = jnp.pad(edge_weight, ((0, pad), (0, 0)))
    attr = jnp.pad(edge_attr, ((0, pad), (0, 0)))

    xj = _make_gather(n, hid, e_pad)(x, jp)
    msg = _make_dense(e_pad, ng, nf, hid)(
        ew, attr, xj, W_mlp1, b_mlp1.reshape(1, nf), W_mlp2,
        b_mlp2.reshape(1, nf), W_lin1)
    scatter, n_pad = _make_scatter(n, nf, e_pad)
    agg = scatter(msg, ip)[:n]
    return _make_final(n, nf, hid)(agg, x, W_lin2, b_lin2.reshape(1, hid))


# speedup vs baseline: 1.1739x; 1.0526x over previous
import math
import functools

import jax
import jax.numpy as jnp
from jax import lax
from jax.experimental import pallas as pl
from jax.experimental.pallas import tpu as pltpu
from jax.experimental.pallas import tpu_sc as plsc

CUTOFF = 10.0

_NC = 2
_NS = 16
_LANES = 16


def _ssp(v):
    return jax.nn.softplus(v) - jnp.log(2.0)


def _make_gather(n_rows_out, d, e_pad):
    nw = _NC * _NS
    per_w = e_pad // nw
    blk = 1024
    n_iter = per_w // blk
    mesh = plsc.VectorSubcoreMesh(core_axis_name="c", subcore_axis_name="s", num_cores=_NC, num_subcores=_NS)

    def body(table_hbm, idx_hbm, out_hbm, idx_v, rows_v, gsem, wsem):
        wid = lax.axis_index("c") * _NS + lax.axis_index("s")
        base = wid * per_w

        def step(b, _):
            off = pl.multiple_of(base + b * blk, blk)
            pltpu.sync_copy(idx_hbm.at[pl.ds(pl.multiple_of(off // 128, 8), 8)],
                            idx_v)
            for i in range(8):
                slot = i & 1
                if i >= 2:
                    pltpu.make_async_copy(
                        rows_v.at[slot],
                        out_hbm.at[pl.ds(off, 128)], wsem).wait()
                else:
                    @pl.when(b > 0)
                    def _():
                        pltpu.make_async_copy(
                            rows_v.at[slot],
                            out_hbm.at[pl.ds(off, 128)], wsem).wait()
                pltpu.async_copy(table_hbm.at[idx_v.at[i]],
                                 rows_v.at[slot], gsem).wait()
                pltpu.async_copy(rows_v.at[slot],
                                 out_hbm.at[pl.ds(off + i * 128, 128)], wsem)
            return 0

        lax.fori_loop(0, n_iter, step, 0)
        for _ in range(2):
            pltpu.make_async_copy(rows_v.at[0],
                                  out_hbm.at[pl.ds(0, 128)], wsem).wait()

    return pl.kernel(
        body,
        out_type=jax.ShapeDtypeStruct((e_pad, d), jnp.float32),
        mesh=mesh,
        scratch_types=[
            pltpu.VMEM((8, 128), jnp.int32),
            pltpu.VMEM((2, 128, d), jnp.float32),
            pltpu.SemaphoreType.DMA,
            pltpu.SemaphoreType.DMA,
        ],
    )


def _xl_body(x_ref, l1_ref, out_ref):
    dn = (((1,), (1,)), ((), ()))
    out_ref[...] = lax.dot_general(x_ref[...], l1_ref[...], dn,
                                   preferred_element_type=jnp.float32)


def _make_xl(n, nf, hid):
    bn = 1000
    return pl.pallas_call(
        _xl_body,
        grid=(n // bn,),
        in_specs=[
            pl.BlockSpec((bn, hid), lambda b: (b, 0)),
            pl.BlockSpec((nf, hid), lambda b: (0, 0)),
        ],
        out_specs=pl.BlockSpec((bn, nf), lambda b: (b, 0)),
        out_shape=jax.ShapeDtypeStruct((n, nf), jnp.float32),
    )


def _dense_body(ew_ref, attr_ref, xlj_ref, w1_ref, b1_ref, w2_ref, b2_ref,
                msg_ref):
    ew = ew_ref[...]
    d2 = jnp.sum(ew * ew, axis=1, keepdims=True)
    d = jnp.sqrt(d2)
    mask = (d <= CUTOFF).astype(jnp.float32)
    cc = 0.5 * (jnp.cos(d * (jnp.pi / CUTOFF)) + 1.0) * mask

    dn = (((1,), (1,)), ((), ()))
    bf = jnp.bfloat16
    h1 = _ssp(
        lax.dot_general(attr_ref[...].astype(bf), w1_ref[...].astype(bf), dn,
                        preferred_element_type=jnp.float32)
        + b1_ref[...]
    )
    w = (lax.dot_general(h1.astype(bf), w2_ref[...].astype(bf), dn,
                         preferred_element_type=jnp.float32)
         + b2_ref[...]) * cc
    msg_ref[...] = w * xlj_ref[...]


def _make_dense(e, e_pad, ng, nf, hid):
    be = 1024
    grid = e_pad // be
    last = (e - 1) // be
    clamp = lambda b: jnp.minimum(b, last)
    full = lambda shape: pl.BlockSpec(shape, lambda b: (0, 0))
    return pl.pallas_call(
        _dense_body,
        grid=(grid,),
        in_specs=[
            pl.BlockSpec((be, 3), lambda b: (clamp(b), 0)),
            pl.BlockSpec((be, ng), lambda b: (clamp(b), 0)),
            pl.BlockSpec((be, nf), lambda b: (b, 0)),
            full((nf, ng)),
            full((1, nf)),
            full((nf, nf)),
            full((1, nf)),
        ],
        out_specs=pl.BlockSpec((be, nf), lambda b: (b, 0)),
        out_shape=jax.ShapeDtypeStruct((e_pad, nf), jnp.float32),
    )


def _make_scatter(n, nf, e_pad):
    nw = _NC * _NS
    rpt = -(-n // (8 * nw)) * 8
    n_pad = rpt * nw
    dummy = rpt
    blk = 1024
    n_blk = e_pad // blk
    ch = 64
    cap = ch + blk + 16

    mesh = plsc.VectorSubcoreMesh(core_axis_name="c", subcore_axis_name="s", num_cores=_NC, num_subcores=_NS)

    def body(msg_hbm, idx_hbm, out_hbm, acc_v, rows_v, idx_v, pd_v, pe_v,
             gsem, isem):
        wid = lax.axis_index("c") * _NS + lax.axis_index("s")
        lo = wid * rpt
        iota = lax.iota(jnp.int32, _LANES)

        def zero_row(r, _):
            for l in range(nf // _LANES):
                acc_v[r, pl.ds(l * _LANES, _LANES)] = jnp.zeros(
                    (_LANES,), jnp.float32)
            return 0

        lax.fori_loop(0, rpt + 1, zero_row, 0)
        for g in range(cap // _LANES):
            pe_v[pl.ds(g * _LANES, _LANES)] = jnp.zeros((_LANES,), jnp.int32)

        def acc_chunk(slot, base):
            def acc_group(g2, _):
                dv = pd_v[pl.ds(base + g2 * _LANES, _LANES)]
                for e in range(_LANES):
                    d = jnp.sum(jnp.where(iota == e, dv, 0))
                    r = g2 * _LANES + e
                    for l in range(nf // _LANES):
                        sl = pl.ds(l * _LANES, _LANES)
                        acc_v[d, sl] = acc_v[d, sl] + rows_v[slot, r, sl]
                return 0

            lax.fori_loop(0, ch // _LANES, acc_group, 0)

        def scan_block(b, w):
            bslot = b & 1
            pltpu.make_async_copy(
                idx_hbm.at[pl.ds(0, 8)], idx_v.at[bslot], isem).wait()

            @pl.when(b + 1 < n_blk)
            def _():
                pltpu.async_copy(
                    idx_hbm.at[pl.ds(
                        pl.multiple_of((b + 1) * (blk // 128), 8), 8)],
                    idx_v.at[(b + 1) & 1], isem)
            for gb in range(0, blk // _LANES, 8):
                vals = []
                for g in range(gb, gb + 8):
                    r, lane = g // 8, (g % 8) * _LANES
                    v = idx_v[bslot, r, pl.ds(lane, _LANES)] - lo
                    ok = (v >= 0) & (v < rpt)
                    vals.append((v, ok, jnp.sum(ok.astype(jnp.int32))))
                for g in range(gb, gb + 8):
                    v, ok, cnt = vals[g - gb]
                    plsc.store_compressed(pd_v.at[pl.ds(w, _LANES)],
                                          jnp.where(ok, v, dummy), mask=ok)
                    eidv = jnp.broadcast_to(b * blk + g * _LANES,
                                            (_LANES,)) + iota
                    plsc.store_compressed(pe_v.at[pl.ds(w, _LANES)], eidv,
                                          mask=ok)
                    w = w + cnt
            nflush = w // ch

            @pl.when(nflush > 0)
            def _():
                pltpu.async_copy(msg_hbm.at[pe_v.at[pl.ds(0, ch)]],
                                 rows_v.at[0], gsem)

            def flush(k, w):
                pltpu.make_async_copy(msg_hbm.at[pe_v.at[pl.ds(0, ch)]],
                                      rows_v.at[0], gsem).wait()

                @pl.when(k + 1 < nflush)
                def _():
                    pltpu.async_copy(
                        msg_hbm.at[pe_v.at[pl.ds(
                            pl.multiple_of((k + 1) * ch, ch), ch)]],
                        rows_v.at[(k + 1) & 1], gsem)

                acc_chunk(k & 1, pl.multiple_of(k * ch, ch))
                return w

            lax.fori_loop(0, nflush, flush, w)
            tail = pl.multiple_of(nflush * ch, ch)
            for g in range(ch // _LANES):
                sl_src = pl.ds(tail + g * _LANES, _LANES)
                sl_dst = pl.ds(g * _LANES, _LANES)
                pd_v[sl_dst] = pd_v[sl_src]
                pe_v[sl_dst] = pe_v[sl_src]
            return w - tail

        pltpu.async_copy(idx_hbm.at[pl.ds(0, 8)], idx_v.at[0], isem)
        w = lax.fori_loop(0, n_blk, scan_block, jnp.int32(0))

        for g in range(ch // _LANES):
            pos = g * _LANES + iota
            dv = pd_v[pl.ds(g * _LANES, _LANES)]
            pd_v[pl.ds(g * _LANES, _LANES)] = jnp.where(pos < w, dv, dummy)
        pltpu.async_copy(msg_hbm.at[pe_v.at[pl.ds(0, ch)]], rows_v.at[0],
                         gsem).wait()
        acc_chunk(0, 0)

        pltpu.sync_copy(acc_v.at[pl.ds(0, rpt)],
                        out_hbm.at[pl.ds(pl.multiple_of(wid * rpt, rpt), rpt)])

    return pl.kernel(
        body,
        out_type=jax.ShapeDtypeStruct((n_pad, nf), jnp.float32),
        mesh=mesh,
        scratch_types=[
            pltpu.VMEM((rpt + 1, nf), jnp.float32),
            pltpu.VMEM((2, ch, nf), jnp.float32),
            pltpu.VMEM((2, 8, 128), jnp.int32),
            pltpu.VMEM((cap,), jnp.int32),
            pltpu.VMEM((cap,), jnp.int32),
            pltpu.SemaphoreType.DMA,
            pltpu.SemaphoreType.DMA,
        ],
        compiler_params=pltpu.CompilerParams(needs_layout_passes=False),
    ), n_pad


def _final_body(agg_ref, x_ref, l2_ref, b2_ref, out_ref):
    dn = (((1,), (1,)), ((), ()))
    h = _ssp(
        lax.dot_general(agg_ref[...], l2_ref[...], dn,
                        preferred_element_type=jnp.float32)
        + b2_ref[...]
    )
    out_ref[...] = x_ref[...] + h


def _make_final(n, nf, hid):
    bn = 1000
    grid = n // bn
    return pl.pallas_call(
        _final_body,
        grid=(grid,),
        in_specs=[
            pl.BlockSpec((bn, nf), lambda b: (b, 0)),
            pl.BlockSpec((bn, hid), lambda b: (b, 0)),
            pl.BlockSpec((hid, nf), lambda b: (0, 0)),
            pl.BlockSpec((1, hid), lambda b: (0, 0)),
        ],
        out_specs=pl.BlockSpec((bn, hid), lambda b: (b, 0)),
        out_shape=jax.ShapeDtypeStruct((n, hid), jnp.float32),
    )


@jax.jit
def kernel(x, edge_index, edge_weight, edge_attr, W_mlp1, b_mlp1, W_mlp2,
           b_mlp2, W_lin1, W_lin2, b_lin2):
    n, hid = x.shape
    e = edge_index.shape[1]
    nf, ng = W_mlp1.shape

    unit = _NC * _NS * 256
    e_pad = -(-e // unit) * unit
    pad = e_pad - e

    i_idx = edge_index[0].astype(jnp.int32)
    j_idx = edge_index[1].astype(jnp.int32)
    ip = jnp.pad(i_idx, (0, pad),
                 constant_values=1 << 29).reshape(e_pad // 128, 128)
    jp = jnp.concatenate(
        [j_idx, jnp.arange(pad, dtype=jnp.int32) % n]
    ).reshape(e_pad // 128, 128)
    xl = _make_xl(n, nf, hid)(x, W_lin1)
    xlj = _make_gather(n, nf, e_pad)(xl, jp)
    msg = _make_dense(e, e_pad, ng, nf, hid)(
        edge_weight, edge_attr, xlj, W_mlp1, b_mlp1.reshape(1, nf), W_mlp2,
        b_mlp2.reshape(1, nf))
    scatter, n_pad = _make_scatter(n, nf, e_pad)
    agg = scatter(msg, ip)[:n]
    return _make_final(n, nf, hid)(agg, x, W_lin2, b_lin2.reshape(1, hid))
